# Initial kernel scaffold; baseline (speedup 1.0000x reference)
import jax
import jax.numpy as jnp
from jax.experimental import pallas as pl


def kernel(hidden_states, W_enc, b_enc, W_dec, b_dec):
    raise NotImplementedError("write your pallas kernel here")


# speedup vs baseline: 1.0001x; 1.0001x over previous
import jax
import jax.numpy as jnp
from jax.experimental import pallas as pl

D_MODEL = 2048
N_FEATURES = 32768
TOKENS = 2048
K = 64
INTERV_IDX = 123
INTERV_VAL = 5.0
EPS = 1e-6


def _norm_body(h_ref, x_ref, mu_ref, sd_ref):
    h = h_ref[...]
    mu = jnp.mean(h, axis=-1, keepdims=True)
    var = jnp.mean((h - mu) ** 2, axis=-1, keepdims=True)
    sd = jnp.sqrt(var)
    x_ref[...] = (h - mu) / (sd + EPS)
    mu_ref[...] = mu
    sd_ref[...] = sd


def kernel(hidden_states, W_enc, b_enc, W_dec, b_dec):
    T, D = hidden_states.shape
    TB = 256
    x, mu, sd = pl.pallas_call(
        _norm_body,
        grid=(T // TB,),
        in_specs=[pl.BlockSpec((TB, D), lambda i: (i, 0))],
        out_specs=[
            pl.BlockSpec((TB, D), lambda i: (i, 0)),
            pl.BlockSpec((TB, 1), lambda i: (i, 0)),
            pl.BlockSpec((TB, 1), lambda i: (i, 0)),
        ],
        out_shape=[
            jax.ShapeDtypeStruct((T, D), jnp.float32),
            jax.ShapeDtypeStruct((T, 1), jnp.float32),
            jax.ShapeDtypeStruct((T, 1), jnp.float32),
        ],
    )(hidden_states)

    pre_acts = jax.nn.relu(x @ W_enc + b_enc)
    feat_vals, feat_idx = jax.lax.top_k(pre_acts, K)

    mask = feat_idx == INTERV_IDX
    feat_vals = jnp.where(mask, INTERV_VAL, feat_vals)
    is_activated = jnp.any(mask, axis=-1)
    min_val = jnp.min(feat_vals, axis=-1)
    min_ind = jnp.argmin(feat_vals, axis=-1)
    tok = jnp.arange(feat_vals.shape[0])
    set_val = jnp.where(is_activated, min_val, INTERV_VAL)
    set_ind = jnp.where(is_activated, feat_idx[tok, min_ind], INTERV_IDX)
    feat_vals = feat_vals.at[tok, min_ind].set(set_val)
    feat_idx = feat_idx.at[tok, min_ind].set(set_ind)

    dec_rows = jnp.take(W_dec, feat_idx, axis=0)
    sae_output = jnp.einsum("tk,tkd->td", feat_vals, dec_rows) + b_dec
    reconstructed = sae_output * (sd + EPS) + mu
    return reconstructed, feat_vals, feat_idx


# speedup vs baseline: 1.0012x; 1.0011x over previous
import functools

import jax
import jax.numpy as jnp
from jax.experimental import pallas as pl
from jax.experimental.pallas import tpu as pltpu

D_MODEL = 2048
N_FEATURES = 32768
TOKENS = 2048
K = 64
INTERV_IDX = 123
INTERV_VAL = 5.0
EPS = 1e-6

FB = 256


def _enc_body(h_ref, w_ref, b_ref, pre_ref, mu_ref, sd_ref, xn_ref):
    j = pl.program_id(0)

    @pl.when(j == 0)
    def _():
        h = h_ref[...]
        mu = jnp.mean(h, axis=-1, keepdims=True)
        var = jnp.mean((h - mu) ** 2, axis=-1, keepdims=True)
        sd = jnp.sqrt(var)
        xn_ref[...] = (h - mu) / (sd + EPS)
        mu_ref[...] = mu
        sd_ref[...] = sd

    acts = jnp.dot(xn_ref[...], w_ref[...], preferred_element_type=jnp.float32)
    pre_ref[...] = jnp.maximum(acts + b_ref[...], 0.0)


def _encode(hidden_states, W_enc, b_enc):
    T, D = hidden_states.shape
    F = W_enc.shape[1]
    pre, mu, sd = pl.pallas_call(
        _enc_body,
        grid=(F // FB,),
        in_specs=[
            pl.BlockSpec((T, D), lambda j: (0, 0)),
            pl.BlockSpec((D, FB), lambda j: (0, j)),
            pl.BlockSpec((1, FB), lambda j: (0, j)),
        ],
        out_specs=[
            pl.BlockSpec((T, FB), lambda j: (0, j)),
            pl.BlockSpec((T, 1), lambda j: (0, 0)),
            pl.BlockSpec((T, 1), lambda j: (0, 0)),
        ],
        out_shape=[
            jax.ShapeDtypeStruct((T, F), jnp.float32),
            jax.ShapeDtypeStruct((T, 1), jnp.float32),
            jax.ShapeDtypeStruct((T, 1), jnp.float32),
        ],
        scratch_shapes=[pltpu.VMEM((T, D), jnp.float32)],
    )(hidden_states, W_enc, b_enc.reshape(1, F))
    return pre, mu, sd


def kernel(hidden_states, W_enc, b_enc, W_dec, b_dec):
    pre_acts, mu, sd = _encode(hidden_states, W_enc, b_enc)

    feat_vals, feat_idx = jax.lax.top_k(pre_acts, K)

    mask = feat_idx == INTERV_IDX
    feat_vals = jnp.where(mask, INTERV_VAL, feat_vals)
    is_activated = jnp.any(mask, axis=-1)
    min_val = jnp.min(feat_vals, axis=-1)
    min_ind = jnp.argmin(feat_vals, axis=-1)
    tok = jnp.arange(feat_vals.shape[0])
    set_val = jnp.where(is_activated, min_val, INTERV_VAL)
    set_ind = jnp.where(is_activated, feat_idx[tok, min_ind], INTERV_IDX)
    feat_vals = feat_vals.at[tok, min_ind].set(set_val)
    feat_idx = feat_idx.at[tok, min_ind].set(set_ind)

    dec_rows = jnp.take(W_dec, feat_idx, axis=0)
    sae_output = jnp.einsum("tk,tkd->td", feat_vals, dec_rows) + b_dec
    reconstructed = sae_output * (sd + EPS) + mu
    return reconstructed, feat_vals, feat_idx


# speedup vs baseline: 8.5435x; 8.5332x over previous
import functools

import jax
import jax.numpy as jnp
from jax import lax
from jax.experimental import pallas as pl
from jax.experimental.pallas import tpu as pltpu
from jax.experimental.pallas import tpu_sc as plsc

D_MODEL = 2048
N_FEATURES = 32768
TOKENS = 2048
K = 64
INTERV_IDX = 123
INTERV_VAL = 5.0
EPS = 1e-6

FB = 256

_NC = 2
_NS = 16
_NW = _NC * _NS
_TPW = TOKENS // _NW
_G = 128
_GS = N_FEATURES // _G
_GV = _GS // 16
_BIG = 1 << 30


def _enc_body(h_ref, w_ref, b_ref, pre_ref, mu_ref, sd_ref, xn_ref):
    j = pl.program_id(0)

    @pl.when(j == 0)
    def _():
        h = h_ref[...]
        mu = jnp.mean(h, axis=-1, keepdims=True)
        var = jnp.mean((h - mu) ** 2, axis=-1, keepdims=True)
        sd = jnp.sqrt(var)
        xn_ref[...] = (h - mu) / (sd + EPS)
        mu_ref[...] = mu
        sd_ref[...] = sd

    acts = jnp.dot(xn_ref[...], w_ref[...], preferred_element_type=jnp.float32)
    pre_ref[...] = jnp.maximum(acts + b_ref[...], 0.0)


def _encode(hidden_states, W_enc, b_enc):
    T, D = hidden_states.shape
    F = W_enc.shape[1]
    pre, mu, sd = pl.pallas_call(
        _enc_body,
        grid=(F // FB,),
        in_specs=[
            pl.BlockSpec((T, D), lambda j: (0, 0)),
            pl.BlockSpec((D, FB), lambda j: (0, j)),
            pl.BlockSpec((1, FB), lambda j: (0, j)),
        ],
        out_specs=[
            pl.BlockSpec((T, FB), lambda j: (0, j)),
            pl.BlockSpec((T, 1), lambda j: (0, 0)),
            pl.BlockSpec((T, 1), lambda j: (0, 0)),
        ],
        out_shape=[
            jax.ShapeDtypeStruct((T, F), jnp.float32),
            jax.ShapeDtypeStruct((T, 1), jnp.float32),
            jax.ShapeDtypeStruct((T, 1), jnp.float32),
        ],
        scratch_shapes=[pltpu.VMEM((T, D), jnp.float32)],
    )(hidden_states, W_enc, b_enc.reshape(1, F))
    return pre, mu, sd


def _iota16():
    return lax.iota(jnp.int32, 16)


def _bf(s):
    return jnp.full((16,), s, dtype=jnp.float32)


def _bi(s):
    return jnp.full((16,), s, dtype=jnp.int32)


def _group_max(row, base):
    m = row[pl.ds(base, 16)]
    for j in range(1, _GV):
        m = jnp.maximum(m, row[pl.ds(base + j * 16, 16)])
    return jnp.max(m)


def _init_sm(row, sm, lane0):
    def body(g, carry):
        s = _group_max(row, g * _GS)
        plsc.store_scatter(sm, [_bi(g)], _bf(s), mask=lane0)
        return carry

    lax.fori_loop(0, _G, body, 0)


def _extract(row, sm, vals_v, idx_v, idx_st, iot, lane0):
    def body(k, carry):
        m = sm[pl.ds(0, 16)]
        for j in range(1, _G // 16):
            m = jnp.maximum(m, sm[pl.ds(j * 16, 16)])
        mx = jnp.max(m)
        gb = _bi(_BIG)
        for j in range(_G // 16):
            e = sm[pl.ds(j * 16, 16)] == mx
            gb = jnp.minimum(gb, jnp.where(e, iot + j * 16, _BIG))
        g = jnp.min(gb)
        base = g * _GS
        pb = _bi(_BIG)
        for j in range(_GV):
            v = row[pl.ds(base + j * 16, 16)]
            pb = jnp.minimum(pb, jnp.where(v == mx, iot + j * 16, _BIG))
        pos = base + jnp.min(pb)
        plsc.store_scatter(vals_v, [_bi(k)], _bf(mx), mask=lane0)
        plsc.store_scatter(idx_v, [_bi(k)], _bi(pos), mask=lane0)
        plsc.store_scatter(idx_st, [_bi(k // 8), _bi(k % 8)], _bi(pos), mask=lane0)
        plsc.store_scatter(row, [_bi(pos)], _bf(-1.0), mask=lane0)
        s2 = _group_max(row, base)
        plsc.store_scatter(sm, [_bi(g)], _bf(s2), mask=lane0)
        return carry

    lax.fori_loop(0, K, body, 0)


def _intervene(vals_v, idx_v, idx_st, iot, lane0):
    cnt = _bi(0)
    for j in range(K // 16):
        ix = idx_v[pl.ds(j * 16, 16)]
        v = vals_v[pl.ds(j * 16, 16)]
        m = ix == INTERV_IDX
        vals_v[pl.ds(j * 16, 16)] = jnp.where(m, INTERV_VAL, v)
        cnt = cnt + jnp.where(m, 1, 0)
    nact = jnp.sum(cnt)
    is_act = nact > 0
    mnv = _bf(1e30)
    for j in range(K // 16):
        mnv = jnp.minimum(mnv, vals_v[pl.ds(j * 16, 16)])
    mn = jnp.min(mnv)
    ab = _bi(_BIG)
    for j in range(K // 16):
        v = vals_v[pl.ds(j * 16, 16)]
        ab = jnp.minimum(ab, jnp.where(v == mn, iot + j * 16, _BIG))
    mi = jnp.min(ab)
    cur = jnp.max(plsc.load_gather(idx_v, [_bi(mi)]))
    set_val = jnp.where(is_act, mn, INTERV_VAL)
    set_ind = jnp.where(is_act, cur, INTERV_IDX)
    plsc.store_scatter(vals_v, [_bi(mi)], _bf(set_val), mask=lane0)
    plsc.store_scatter(idx_v, [_bi(mi)], _bi(set_ind), mask=lane0)
    plsc.store_scatter(idx_st, [_bi(mi // 8), _bi(mi % 8)], _bi(set_ind), mask=lane0)


def _decode(tl, t, vals_v, idx_st, gbufs, gsems, acc, bdv, muv, sdv, wdec,
            rec_o, iot):
    pltpu.make_async_copy(wdec.at[idx_st.at[0]], gbufs[0], gsems[0]).start()
    for c in range(8):
        if c < 7:
            pltpu.make_async_copy(
                wdec.at[idx_st.at[c + 1]], gbufs[(c + 1) % 2], gsems[(c + 1) % 2]
            ).start()
        pltpu.make_async_copy(
            wdec.at[idx_st.at[c]], gbufs[c % 2], gsems[c % 2]
        ).wait()
        off = 8 * c if c < 7 else K - 16
        lo = 0 if c < 7 else 8
        wv = vals_v[pl.ds(off, 16)]
        ws = [jnp.max(jnp.where(iot == (lo + r), wv, -1e30)) for r in range(8)]
        gb = gbufs[c % 2]
        if c == 0:
            def dbody(d, carry):
                s = gb[0, pl.ds(d * 16, 16)] * ws[0]
                for r in range(1, 8):
                    s = s + gb[r, pl.ds(d * 16, 16)] * ws[r]
                acc[pl.ds(d * 16, 16)] = s
                return carry
        else:
            def dbody(d, carry):
                s = acc[pl.ds(d * 16, 16)]
                for r in range(8):
                    s = s + gb[r, pl.ds(d * 16, 16)] * ws[r]
                acc[pl.ds(d * 16, 16)] = s
                return carry
        lax.fori_loop(0, D_MODEL // 16, dbody, 0)

    mu_s = jnp.max(plsc.load_gather(muv, [_bi(tl)]))
    sd_s = jnp.max(plsc.load_gather(sdv, [_bi(tl)]))
    scale = sd_s + EPS

    def fbody(d, carry):
        acc[pl.ds(d * 16, 16)] = (
            acc[pl.ds(d * 16, 16)] + bdv[pl.ds(d * 16, 16)]
        ) * scale + mu_s
        return carry

    lax.fori_loop(0, D_MODEL // 16, fbody, 0)
    pltpu.sync_copy(acc, rec_o.at[t])


def _sc_body(pre, mu, sd, wdec, bdec,
             vals_o, idx_o, rec_o,
             rowb0, rowb1, sm, vals_v, idx_v, idx_st, gb0, gb1, acc,
             bdv, muv, sdv, rs0, rs1, gs0, gs1):
    wid = lax.axis_index("s") * _NC + lax.axis_index("c")
    base_t = wid * _TPW
    iot = _iota16()
    lane0 = iot == 0
    pltpu.sync_copy(bdec, bdv)
    pltpu.sync_copy(mu.at[pl.ds(base_t, _TPW)], muv)
    pltpu.sync_copy(sd.at[pl.ds(base_t, _TPW)], sdv)
    pltpu.make_async_copy(pre.at[base_t], rowb0, rs0).start()
    pltpu.make_async_copy(pre.at[base_t + 1], rowb1, rs1).start()
    rowbs = (rowb0, rowb1)
    rsems = (rs0, rs1)
    gbufs = (gb0, gb1)
    gsems = (gs0, gs1)

    def iter_body(i, carry):
        for p in range(2):
            tl = i * 2 + p
            t = base_t + tl
            row = rowbs[p]
            pltpu.make_async_copy(pre.at[t], row, rsems[p]).wait()
            _init_sm(row, sm, lane0)
            _extract(row, sm, vals_v, idx_v, idx_st, iot, lane0)

            @pl.when(i < _TPW // 2 - 1)
            def _():
                pltpu.make_async_copy(pre.at[t + 2], row, rsems[p]).start()

            _intervene(vals_v, idx_v, idx_st, iot, lane0)
            pltpu.sync_copy(vals_v, vals_o.at[t])
            pltpu.sync_copy(idx_v, idx_o.at[t])
            _decode(tl, t, vals_v, idx_st, gbufs, gsems, acc, bdv, muv, sdv,
                    wdec, rec_o, iot)
        return carry

    lax.fori_loop(0, _TPW // 2, iter_body, 0)


def _sae_sc(pre, mu, sd, W_dec, b_dec):
    mesh = plsc.VectorSubcoreMesh(core_axis_name="c", subcore_axis_name="s")
    f = functools.partial(
        pl.kernel,
        mesh=mesh,
        compiler_params=pltpu.CompilerParams(needs_layout_passes=False),
        out_type=[
            jax.ShapeDtypeStruct((TOKENS, K), jnp.float32),
            jax.ShapeDtypeStruct((TOKENS, K), jnp.int32),
            jax.ShapeDtypeStruct((TOKENS, D_MODEL), jnp.float32),
        ],
        scratch_types=[
            pltpu.VMEM((N_FEATURES,), jnp.float32),
            pltpu.VMEM((N_FEATURES,), jnp.float32),
            pltpu.VMEM((_G,), jnp.float32),
            pltpu.VMEM((K,), jnp.float32),
            pltpu.VMEM((K,), jnp.int32),
            pltpu.VMEM((8, 8), jnp.int32),
            pltpu.VMEM((8, D_MODEL), jnp.float32),
            pltpu.VMEM((8, D_MODEL), jnp.float32),
            pltpu.VMEM((D_MODEL,), jnp.float32),
            pltpu.VMEM((D_MODEL,), jnp.float32),
            pltpu.VMEM((_TPW,), jnp.float32),
            pltpu.VMEM((_TPW,), jnp.float32),
            pltpu.SemaphoreType.DMA,
            pltpu.SemaphoreType.DMA,
            pltpu.SemaphoreType.DMA,
            pltpu.SemaphoreType.DMA,
        ],
    )(_sc_body)
    return f(pre, mu, sd, W_dec, b_dec)


def kernel(hidden_states, W_enc, b_enc, W_dec, b_dec):
    pre_acts, mu, sd = _encode(hidden_states, W_enc, b_enc)
    vals, idx, rec = _sae_sc(
        pre_acts, mu.reshape(-1), sd.reshape(-1), W_dec, b_dec
    )
    return rec, vals, idx


# speedup vs baseline: 8.6253x; 1.0096x over previous
import functools

import jax
import jax.numpy as jnp
from jax import lax
from jax.experimental import pallas as pl
from jax.experimental.pallas import tpu as pltpu
from jax.experimental.pallas import tpu_sc as plsc

D_MODEL = 2048
N_FEATURES = 32768
TOKENS = 2048
K = 64
INTERV_IDX = 123
INTERV_VAL = 5.0
EPS = 1e-6

FB = 256

_NC = 2
_NS = 16
_NW = _NC * _NS
_TPW = TOKENS // _NW
_G = 128
_GS = N_FEATURES // _G
_GV = _GS // 16
_BIG = 1 << 30


def _enc_body(h_ref, w_ref, b_ref, pre_ref, mu_ref, sd_ref, xn_ref):
    j = pl.program_id(0)

    @pl.when(j == 0)
    def _():
        h = h_ref[...]
        mu = jnp.mean(h, axis=-1, keepdims=True)
        var = jnp.mean((h - mu) ** 2, axis=-1, keepdims=True)
        sd = jnp.sqrt(var)
        xn_ref[...] = (h - mu) / (sd + EPS)
        mu_ref[...] = mu
        sd_ref[...] = sd

    acts = jnp.dot(xn_ref[...], w_ref[...], preferred_element_type=jnp.float32)
    pre_ref[...] = jnp.maximum(acts + b_ref[...], 0.0)


def _encode(hidden_states, W_enc, b_enc):
    T, D = hidden_states.shape
    F = W_enc.shape[1]
    pre, mu, sd = pl.pallas_call(
        _enc_body,
        grid=(F // FB,),
        in_specs=[
            pl.BlockSpec((T, D), lambda j: (0, 0)),
            pl.BlockSpec((D, FB), lambda j: (0, j)),
            pl.BlockSpec((1, FB), lambda j: (0, j)),
        ],
        out_specs=[
            pl.BlockSpec((T, FB), lambda j: (0, j)),
            pl.BlockSpec((T, 1), lambda j: (0, 0)),
            pl.BlockSpec((T, 1), lambda j: (0, 0)),
        ],
        out_shape=[
            jax.ShapeDtypeStruct((T, F), jnp.float32),
            jax.ShapeDtypeStruct((T, 1), jnp.float32),
            jax.ShapeDtypeStruct((T, 1), jnp.float32),
        ],
        scratch_shapes=[pltpu.VMEM((T, D), jnp.float32)],
    )(hidden_states, W_enc, b_enc.reshape(1, F))
    return pre, mu, sd


def _iota16():
    return lax.iota(jnp.int32, 16)


def _bf(s):
    return jnp.full((16,), s, dtype=jnp.float32)


def _bi(s):
    return jnp.full((16,), s, dtype=jnp.int32)


def _group_max(row, base):
    m = row[pl.ds(base, 16)]
    for j in range(1, _GV):
        m = jnp.maximum(m, row[pl.ds(base + j * 16, 16)])
    return jnp.max(m)


def _init_sm(row, sm, lane0):
    def body(g, carry):
        s = _group_max(row, g * _GS)
        plsc.store_scatter(sm, [_bi(g)], _bf(s), mask=lane0)
        return carry

    lax.fori_loop(0, _G, body, 0)


def _extract(row, sm, vals_v, idx_v, idx_st, iot, lane0):
    def body(k, carry):
        m = sm[pl.ds(0, 16)]
        for j in range(1, _G // 16):
            m = jnp.maximum(m, sm[pl.ds(j * 16, 16)])
        mx = jnp.max(m)
        gb = _bi(_BIG)
        for j in range(_G // 16):
            e = sm[pl.ds(j * 16, 16)] == mx
            gb = jnp.minimum(gb, jnp.where(e, iot + j * 16, _BIG))
        g = jnp.min(gb)
        base = g * _GS
        pb = _bi(_BIG)
        for j in range(_GV):
            v = row[pl.ds(base + j * 16, 16)]
            pb = jnp.minimum(pb, jnp.where(v == mx, iot + j * 16, _BIG))
        pos = base + jnp.min(pb)
        plsc.store_scatter(vals_v, [_bi(k)], _bf(mx), mask=lane0)
        plsc.store_scatter(idx_v, [_bi(k)], _bi(pos), mask=lane0)
        plsc.store_scatter(idx_st, [_bi(k // 8), _bi(k % 8)], _bi(pos), mask=lane0)
        plsc.store_scatter(row, [_bi(pos)], _bf(-1.0), mask=lane0)
        s2 = _group_max(row, base)
        plsc.store_scatter(sm, [_bi(g)], _bf(s2), mask=lane0)
        return carry

    lax.fori_loop(0, K, body, 0)


def _intervene(vals_v, idx_v, idx_st, iot, lane0):
    cnt = _bi(0)
    for j in range(K // 16):
        ix = idx_v[pl.ds(j * 16, 16)]
        v = vals_v[pl.ds(j * 16, 16)]
        m = ix == INTERV_IDX
        vals_v[pl.ds(j * 16, 16)] = jnp.where(m, INTERV_VAL, v)
        cnt = cnt + jnp.where(m, 1, 0)
    nact = jnp.sum(cnt)
    is_act = nact > 0
    mnv = _bf(1e30)
    for j in range(K // 16):
        mnv = jnp.minimum(mnv, vals_v[pl.ds(j * 16, 16)])
    mn = jnp.min(mnv)
    ab = _bi(_BIG)
    for j in range(K // 16):
        v = vals_v[pl.ds(j * 16, 16)]
        ab = jnp.minimum(ab, jnp.where(v == mn, iot + j * 16, _BIG))
    mi = jnp.min(ab)
    cur = jnp.max(plsc.load_gather(idx_v, [_bi(mi)]))
    set_val = jnp.where(is_act, mn, INTERV_VAL)
    set_ind = jnp.where(is_act, cur, INTERV_IDX)
    plsc.store_scatter(vals_v, [_bi(mi)], _bf(set_val), mask=lane0)
    plsc.store_scatter(idx_v, [_bi(mi)], _bi(set_ind), mask=lane0)
    plsc.store_scatter(idx_st, [_bi(mi // 8), _bi(mi % 8)], _bi(set_ind), mask=lane0)


def _decode(tl, t, vals_v, idx_st, gbufs, gsems, acc, bdv, muv, sdv, wdec,
            rec_o, iot):
    pltpu.make_async_copy(wdec.at[idx_st.at[0]], gbufs[0], gsems[0]).start()
    for c in range(8):
        if c < 7:
            pltpu.make_async_copy(
                wdec.at[idx_st.at[c + 1]], gbufs[(c + 1) % 2], gsems[(c + 1) % 2]
            ).start()
        pltpu.make_async_copy(
            wdec.at[idx_st.at[c]], gbufs[c % 2], gsems[c % 2]
        ).wait()
        off = 8 * c if c < 7 else K - 16
        lo = 0 if c < 7 else 8
        wv = vals_v[pl.ds(off, 16)]
        ws = [jnp.max(jnp.where(iot == (lo + r), wv, -1e30)) for r in range(8)]
        gb = gbufs[c % 2]
        if c == 0:
            def dbody(d, carry):
                s = gb[0, pl.ds(d * 16, 16)] * ws[0]
                for r in range(1, 8):
                    s = s + gb[r, pl.ds(d * 16, 16)] * ws[r]
                acc[pl.ds(d * 16, 16)] = s
                return carry
        else:
            def dbody(d, carry):
                s = acc[pl.ds(d * 16, 16)]
                for r in range(8):
                    s = s + gb[r, pl.ds(d * 16, 16)] * ws[r]
                acc[pl.ds(d * 16, 16)] = s
                return carry
        lax.fori_loop(0, D_MODEL // 16, dbody, 0, unroll=4)

    mu_s = jnp.max(plsc.load_gather(muv, [_bi(tl)]))
    sd_s = jnp.max(plsc.load_gather(sdv, [_bi(tl)]))
    scale = sd_s + EPS

    def fbody(d, carry):
        acc[pl.ds(d * 16, 16)] = (
            acc[pl.ds(d * 16, 16)] + bdv[pl.ds(d * 16, 16)]
        ) * scale + mu_s
        return carry

    lax.fori_loop(0, D_MODEL // 16, fbody, 0, unroll=4)
    pltpu.sync_copy(acc, rec_o.at[t])


def _sc_body(pre, mu, sd, wdec, bdec,
             vals_o, idx_o, rec_o,
             rowb0, rowb1, sm, vals_v, idx_v, idx_st, gb0, gb1, acc,
             bdv, muv, sdv, rs0, rs1, gs0, gs1):
    wid = lax.axis_index("s") * _NC + lax.axis_index("c")
    base_t = wid * _TPW
    iot = _iota16()
    lane0 = iot == 0
    pltpu.sync_copy(bdec, bdv)
    pltpu.sync_copy(mu.at[pl.ds(base_t, _TPW)], muv)
    pltpu.sync_copy(sd.at[pl.ds(base_t, _TPW)], sdv)
    pltpu.make_async_copy(pre.at[base_t], rowb0, rs0).start()
    pltpu.make_async_copy(pre.at[base_t + 1], rowb1, rs1).start()
    rowbs = (rowb0, rowb1)
    rsems = (rs0, rs1)
    gbufs = (gb0, gb1)
    gsems = (gs0, gs1)

    def iter_body(i, carry):
        for p in range(2):
            tl = i * 2 + p
            t = base_t + tl
            row = rowbs[p]
            pltpu.make_async_copy(pre.at[t], row, rsems[p]).wait()
            _init_sm(row, sm, lane0)
            _extract(row, sm, vals_v, idx_v, idx_st, iot, lane0)

            @pl.when(i < _TPW // 2 - 1)
            def _():
                pltpu.make_async_copy(pre.at[t + 2], row, rsems[p]).start()

            _intervene(vals_v, idx_v, idx_st, iot, lane0)
            pltpu.sync_copy(vals_v, vals_o.at[t])
            pltpu.sync_copy(idx_v, idx_o.at[t])
            _decode(tl, t, vals_v, idx_st, gbufs, gsems, acc, bdv, muv, sdv,
                    wdec, rec_o, iot)
        return carry

    lax.fori_loop(0, _TPW // 2, iter_body, 0)


def _sae_sc(pre, mu, sd, W_dec, b_dec):
    mesh = plsc.VectorSubcoreMesh(core_axis_name="c", subcore_axis_name="s")
    f = functools.partial(
        pl.kernel,
        mesh=mesh,
        compiler_params=pltpu.CompilerParams(needs_layout_passes=False),
        out_type=[
            jax.ShapeDtypeStruct((TOKENS, K), jnp.float32),
            jax.ShapeDtypeStruct((TOKENS, K), jnp.int32),
            jax.ShapeDtypeStruct((TOKENS, D_MODEL), jnp.float32),
        ],
        scratch_types=[
            pltpu.VMEM((N_FEATURES,), jnp.float32),
            pltpu.VMEM((N_FEATURES,), jnp.float32),
            pltpu.VMEM((_G,), jnp.float32),
            pltpu.VMEM((K,), jnp.float32),
            pltpu.VMEM((K,), jnp.int32),
            pltpu.VMEM((8, 8), jnp.int32),
            pltpu.VMEM((8, D_MODEL), jnp.float32),
            pltpu.VMEM((8, D_MODEL), jnp.float32),
            pltpu.VMEM((D_MODEL,), jnp.float32),
            pltpu.VMEM((D_MODEL,), jnp.float32),
            pltpu.VMEM((_TPW,), jnp.float32),
            pltpu.VMEM((_TPW,), jnp.float32),
            pltpu.SemaphoreType.DMA,
            pltpu.SemaphoreType.DMA,
            pltpu.SemaphoreType.DMA,
            pltpu.SemaphoreType.DMA,
        ],
    )(_sc_body)
    return f(pre, mu, sd, W_dec, b_dec)


def kernel(hidden_states, W_enc, b_enc, W_dec, b_dec):
    pre_acts, mu, sd = _encode(hidden_states, W_enc, b_enc)
    vals, idx, rec = _sae_sc(
        pre_acts, mu.reshape(-1), sd.reshape(-1), W_dec, b_dec
    )
    return rec, vals, idx
